# Initial kernel scaffold; baseline (speedup 1.0000x reference)
import jax
import jax.numpy as jnp
from jax.experimental import pallas as pl


def kernel(regressions, anchors, annotations):
    raise NotImplementedError("write your pallas kernel here")


# speedup vs baseline: 11.3287x; 11.3287x over previous
import functools

import jax
import jax.numpy as jnp
from jax import lax
from jax.experimental import pallas as pl
from jax.experimental.pallas import tpu as pltpu

N_PAD = 20480
ROWS = 160
ROW_BLK = 16
N_BLKS = ROWS // ROW_BLK
B = 8
M = 32


def _main_body(a_ref, reg_ref, ann_ref, out_ref):
    b = pl.program_id(0)
    ay1 = a_ref[0]
    ax1 = a_ref[1]
    ay2 = a_ref[2]
    ax2 = a_ref[3]
    aw = ax2 - ax1
    ah = ay2 - ay1
    acx = ax1 + 0.5 * aw
    acy = ay1 + 0.5 * ah
    area_a = ah * aw
    law = jnp.log(aw)
    lah = jnp.log(ah)

    zeros = jnp.zeros_like(aw)

    def m_step(m, carry):
        best_inter, best_ua, scx, scy, slw, slh = carry
        bx1 = ann_ref[b, m, 0]
        by1 = ann_ref[b, m, 1]
        bx2 = ann_ref[b, m, 2]
        by2 = ann_ref[b, m, 3]
        bw = bx2 - bx1
        bh = by2 - by1
        area_b = bw * bh
        gcx = bx1 + 0.5 * bw
        gcy = by1 + 0.5 * bh
        glw = jnp.log(jnp.maximum(bw, 1.0))
        glh = jnp.log(jnp.maximum(bh, 1.0))
        iw = jnp.maximum(jnp.minimum(ax2, bx2) - jnp.maximum(ax1, bx1), 0.0)
        ih = jnp.maximum(jnp.minimum(ay2, by2) - jnp.maximum(ay1, by1), 0.0)
        inter = iw * ih
        ua = jnp.maximum(area_a + area_b - inter, 1e-8)
        cond = inter * best_ua > best_inter * ua
        best_inter = jnp.where(cond, inter, best_inter)
        best_ua = jnp.where(cond, ua, best_ua)
        scx = jnp.where(cond, gcx, scx)
        scy = jnp.where(cond, gcy, scy)
        slw = jnp.where(cond, glw, slw)
        slh = jnp.where(cond, glh, slh)
        return best_inter, best_ua, scx, scy, slw, slh

    init = (zeros - 1.0, zeros + 1.0, zeros, zeros, zeros, zeros)
    best_inter, best_ua, scx, scy, slw, slh = lax.fori_loop(0, M, m_step, init)

    positive = best_inter * 2.0 >= best_ua
    dy = (scy - acy) / ah
    dx = (scx - acx) / aw
    dh = slh - lah
    dw = slw - law
    loss = zeros
    for j, t in enumerate((dy, dx, dh, dw)):
        diff = jnp.abs(t - reg_ref[0, j])
        loss += jnp.where(diff <= 1.0 / 9.0, 4.5 * diff * diff, diff - 0.5 / 9.0)
    loss = jnp.where(positive, loss, 0.0)

    @pl.when(pl.program_id(1) == 0)
    def _():
        out_ref[b, 0] = 0.0
        out_ref[b, 1] = 0.0

    out_ref[b, 0] += jnp.sum(loss)
    out_ref[b, 1] += jnp.sum(positive.astype(jnp.float32))


def _combine_body(p_ref, out_ref):
    total = 0.0
    for b in range(B):
        ls = p_ref[b, 0]
        np_ = p_ref[b, 1]
        total += jnp.where(np_ > 0.0, ls / (4.0 * jnp.maximum(np_, 1.0)), 0.0)
    out_ref[0, 0] = total * (50.0 / B)


@jax.jit
def kernel(regressions, anchors, annotations):
    n = anchors.shape[1]
    pad_box = jnp.array([-1e4, -1e4, -1e4 + 1.0, -1e4 + 1.0], jnp.float32)
    a = jnp.concatenate(
        [anchors[0], jnp.broadcast_to(pad_box, (N_PAD - n, 4))], axis=0
    )
    a_t = a.T.reshape(4, ROWS, 128)
    reg = jnp.concatenate(
        [regressions, jnp.zeros((B, N_PAD - n, 4), jnp.float32)], axis=1
    )
    reg_t = reg.transpose(0, 2, 1).reshape(B, 4, ROWS, 128)

    partials = pl.pallas_call(
        _main_body,
        grid=(B, N_BLKS),
        in_specs=[
            pl.BlockSpec((4, ROW_BLK, 128), lambda b, i: (0, i, 0)),
            pl.BlockSpec((1, 4, ROW_BLK, 128), lambda b, i: (b, 0, i, 0)),
            pl.BlockSpec(memory_space=pltpu.SMEM),
        ],
        out_specs=pl.BlockSpec(memory_space=pltpu.SMEM),
        out_shape=jax.ShapeDtypeStruct((B, 2), jnp.float32),
    )(a_t, reg_t, annotations)

    out = pl.pallas_call(
        _combine_body,
        in_specs=[pl.BlockSpec(memory_space=pltpu.SMEM)],
        out_specs=pl.BlockSpec(memory_space=pltpu.SMEM),
        out_shape=jax.ShapeDtypeStruct((1, 1), jnp.float32),
    )(partials)
    return out.reshape(1)
